# Initial kernel scaffold; baseline (speedup 1.0000x reference)
import jax
import jax.numpy as jnp
from jax.experimental import pallas as pl


def kernel(x, edge_index, batch, node, di_sim, dr_sim, drug_adj, dis_adj, p, W1, b1, W2, b2, conv1_w, conv1_b, conv2_w, conv2_b, lin1_W, lin1_b, lin2_W, lin2_b, bn_gamma, bn_beta, lin_di_W, lin_di_b, lin_dr_W, lin_dr_b, W3, b3, W4, b4, fcs_W, fcs_b, fcs2_W, fcs2_b):
    raise NotImplementedError("write your pallas kernel here")


# speedup vs baseline: 17.0286x; 17.0286x over previous
import functools

import jax
import jax.numpy as jnp
from jax import lax
from jax.experimental import pallas as pl
from jax.experimental.pallas import tpu as pltpu
from jax.experimental.pallas import tpu_sc as plsc

N = 10000
NPAD = 10496
B = 256
K = 30
W = 256
E = 320000
NC, NS = 2, 16
NW = NC * NS
C = 128
NCH = 80
EPW = NCH * C
E_PAD = NW * EPW
RPT = NPAD // NS
DEG_W = 16
F1 = 32

@functools.cache
def _sc_kernels():
    mesh = plsc.VectorSubcoreMesh(core_axis_name="c", subcore_axis_name="s",
                                  num_cores=NC, num_subcores=NS)

    def _worker_id():
        return lax.axis_index("s") * NC + lax.axis_index("c")

    @functools.partial(
        pl.kernel,
        out_type=(
            jax.ShapeDtypeStruct((NPAD, DEG_W), jnp.float32),
            jax.ShapeDtypeStruct((NPAD, DEG_W), jnp.float32),
        ),
        mesh=mesh,
        scratch_types=[
            pltpu.VMEM((NCH, C), jnp.int32),
            pltpu.VMEM((C, DEG_W), jnp.float32),
            pltpu.VMEM_SHARED((NPAD, DEG_W), jnp.float32),
        ],
        compiler_params=pltpu.CompilerParams(use_tc_tiling_on_sc=False),
    )
    def _sc_degree(dst_hbm, ones_hbm, zeros_hbm, out0, out1,
                   idx_v, ones_v, acc_sh):
        cid = lax.axis_index("c")
        sid = lax.axis_index("s")
        wid = _worker_id()
        pltpu.sync_copy(dst_hbm.at[pl.ds(wid * NCH, NCH), :], idx_v)
        pltpu.sync_copy(ones_hbm, ones_v)
        pltpu.sync_copy(zeros_hbm, acc_sh.at[pl.ds(sid * RPT, RPT), :])
        plsc.subcore_barrier()

        def body(j, carry):
            pltpu.sync_copy(ones_v, acc_sh.at[idx_v.at[j]], add=True)
            return carry

        lax.fori_loop(0, NCH, body, 0)
        plsc.subcore_barrier()

        @pl.when(cid == 0)
        def _():
            pltpu.sync_copy(acc_sh.at[pl.ds(sid * RPT, RPT), :],
                            out0.at[pl.ds(sid * RPT, RPT), :])

        @pl.when(cid == 1)
        def _():
            pltpu.sync_copy(acc_sh.at[pl.ds(sid * RPT, RPT), :],
                            out1.at[pl.ds(sid * RPT, RPT), :])

    @functools.partial(
        pl.kernel,
        out_type=(
            jax.ShapeDtypeStruct((NPAD, F1), jnp.float32),
            jax.ShapeDtypeStruct((NPAD, F1), jnp.float32),
        ),
        mesh=mesh,
        scratch_types=[
            pltpu.VMEM((NCH, C), jnp.int32),
            pltpu.VMEM((NCH, C), jnp.int32),
            pltpu.VMEM((C, F1), jnp.float32),
            pltpu.VMEM_SHARED((NPAD, F1), jnp.float32),
            pltpu.SemaphoreType.DMA,
        ],
        compiler_params=pltpu.CompilerParams(use_tc_tiling_on_sc=False),
    )
    def _sc_aggregate(src_hbm, dst_hbm, hs_hbm, zeros_hbm, out0, out1,
                      src_v, dst_v, rows_v, acc_sh, sem):
        cid = lax.axis_index("c")
        sid = lax.axis_index("s")
        wid = _worker_id()
        pltpu.sync_copy(src_hbm.at[pl.ds(wid * NCH, NCH), :], src_v)
        pltpu.sync_copy(dst_hbm.at[pl.ds(wid * NCH, NCH), :], dst_v)
        pltpu.sync_copy(zeros_hbm, acc_sh.at[pl.ds(sid * RPT, RPT), :])
        plsc.subcore_barrier()

        def body(j, carry):
            pltpu.async_copy(hs_hbm.at[src_v.at[j]], rows_v, sem).wait()
            pltpu.sync_copy(rows_v, acc_sh.at[dst_v.at[j]], add=True)
            return carry

        lax.fori_loop(0, NCH, body, 0)
        plsc.subcore_barrier()

        @pl.when(cid == 0)
        def _():
            pltpu.sync_copy(acc_sh.at[pl.ds(sid * RPT, RPT), :],
                            out0.at[pl.ds(sid * RPT, RPT), :])

        @pl.when(cid == 1)
        def _():
            pltpu.sync_copy(acc_sh.at[pl.ds(sid * RPT, RPT), :],
                            out1.at[pl.ds(sid * RPT, RPT), :])

    return _sc_degree, _sc_aggregate


def _dense_gcn_pair(emb, adj, W3, b3, W4, b4):
    n = adj.shape[0]
    i0 = lax.broadcasted_iota(jnp.int32, (n, n), 0)
    i1 = lax.broadcasted_iota(jnp.int32, (n, n), 1)
    a = adj + jnp.where(i0 == i1, 1.0, 0.0).astype(jnp.float32)
    deg = jnp.sum(a, axis=1, keepdims=True)
    dinv = lax.rsqrt(deg)
    f = jax.nn.relu(dinv * jnp.dot(a, dinv * jnp.dot(emb, W3),
                                   preferred_element_type=jnp.float32) + b3)
    o = jax.nn.relu(dinv * jnp.dot(a, dinv * jnp.dot(f, W4),
                                   preferred_element_type=jnp.float32) + b4)
    return o


def _tc_prep_body(x_ref, w1_ref, d0_ref, d1_ref, batch_ref, node_ref,
                  disim_ref, ldiW_ref, ldib_ref, dradj_ref,
                  drsim_ref, ldrW_ref, ldrb_ref, disadj_ref,
                  w3_ref, b3_ref, w4_ref, b4_ref,
                  h1_ref, hs1_ref, dinv_ref, dinv2_ref, feat_ref,
                  starts_ref, counts_ref):
    deg = jnp.sum(d0_ref[...] + d1_ref[...], axis=1,
                  keepdims=True) * (1.0 / DEG_W) + 1.0
    dinv = lax.rsqrt(deg)
    dinv_ref[...] = dinv
    dinv2_ref[...] = dinv * dinv
    h1 = jnp.dot(x_ref[...], w1_ref[...], preferred_element_type=jnp.float32)
    h1_ref[...] = h1
    hs1_ref[...] = h1 * dinv

    bcb = jnp.broadcast_to(batch_ref[...], (B, NPAD))
    sseg = lax.broadcasted_iota(jnp.int32, (B, NPAD), 0)
    counts_ref[...] = jnp.sum((bcb == sseg).astype(jnp.int32), axis=1,
                              keepdims=True)
    starts_ref[...] = jnp.sum((bcb < sseg).astype(jnp.int32), axis=1,
                              keepdims=True)

    di_emb = jnp.dot(disim_ref[...], ldiW_ref[...],
                     preferred_element_type=jnp.float32) + ldib_ref[...]
    di_out = _dense_gcn_pair(di_emb, dradj_ref[...], w3_ref[...], b3_ref[...],
                             w4_ref[...], b4_ref[...])
    dr_emb = jnp.dot(drsim_ref[...], ldrW_ref[...],
                     preferred_element_type=jnp.float32) + ldrb_ref[...]
    dr_out = _dense_gcn_pair(dr_emb, disadj_ref[...], w3_ref[...], b3_ref[...],
                             w4_ref[...], b4_ref[...])
    node = node_ref[...]
    oh_di = (lax.broadcasted_iota(jnp.int32, (B, 593), 1)
             == node[:, 0:1]).astype(jnp.float32)
    oh_dr = (lax.broadcasted_iota(jnp.int32, (B, 313), 1)
             == node[:, 1:2]).astype(jnp.float32)
    di_sel = jnp.dot(oh_di, di_out, preferred_element_type=jnp.float32)
    dr_sel = jnp.dot(oh_dr, dr_out, preferred_element_type=jnp.float32)
    feat_ref[...] = jnp.concatenate([di_sel, dr_sel], axis=1)


def _tc_mid_body(a0_ref, a1_ref, h1_ref, dinv_ref, dinv2_ref, b1_ref, w2_ref,
                 x1_ref, h2_ref, hs2_ref):
    dinv = dinv_ref[...]
    x1 = jax.nn.relu(dinv * (a0_ref[...] + a1_ref[...])
                     + dinv2_ref[...] * h1_ref[...] + b1_ref[...])
    x1_ref[...] = x1
    h2 = jnp.dot(x1, w2_ref[...], preferred_element_type=jnp.float32)
    h2_ref[...] = h2
    hs2_ref[...] = h2 * dinv


def _tc_pool_body(starts_ref, counts_ref, x1_ref, h2_ref, a0_ref, a1_ref,
                  dinv_ref, dinv2_ref, b2_ref, g_ref):
    s = pl.program_id(0)
    st = starts_ref[s, 0]
    cnt = counts_ref[s, 0]
    x1_w = x1_ref[pl.ds(st, W), :]
    a_w = a0_ref[pl.ds(st, W), :] + a1_ref[pl.ds(st, W), :]
    h2_w = h2_ref[pl.ds(st, W), :]
    dv = dinv_ref[pl.ds(st, W), :]
    dv2 = dinv2_ref[pl.ds(st, W), :]
    x2_w = jax.nn.relu(dv * a_w + dv2 * h2_w + b2_ref[...])
    keys = x2_w[:, F1 - 1:F1]
    krow = jnp.transpose(keys)
    mi = lax.broadcasted_iota(jnp.int32, (W, W), 0)
    ji = lax.broadcasted_iota(jnp.int32, (W, W), 1)
    better = ((keys > krow) | ((keys == krow) & (mi < ji))) & (mi < cnt)
    rank = jnp.sum(better.astype(jnp.float32), axis=0, keepdims=True)
    pk = lax.broadcasted_iota(jnp.int32, (K, W), 0)
    jk = lax.broadcasted_iota(jnp.int32, (K, W), 1)
    sel = ((pk == rank.astype(jnp.int32)) & (jk < cnt)).astype(jnp.float32)
    h_cat = jnp.concatenate([x1_w, x2_w], axis=1)
    g_ref[0] = jnp.dot(sel, h_cat, preferred_element_type=jnp.float32)


def _tc_head_body(g_ref, w1c_ref, b1c_ref, w2r_ref, b2c_ref, l1w_ref, l1b_ref,
                  l2w_ref, l2b_ref, bng_ref, bnb_ref, feat_ref, p_ref,
                  fw_ref, fb_ref, f2w_ref, f2b_ref, out_ref):
    g2 = g_ref[...].reshape(B * K, 64)
    y = jax.nn.relu(jnp.dot(g2, w1c_ref[...],
                            preferred_element_type=jnp.float32) + b1c_ref[...])
    y4 = y.reshape(B, K // 2, 2, 128)
    pooled = jnp.max(y4, axis=2)
    u = jnp.concatenate([pooled[:, k:k + 11, :] for k in range(5)], axis=2)
    u2 = u.reshape(B * 11, 640)
    o2 = jax.nn.relu(jnp.dot(u2, w2r_ref[...],
                             preferred_element_type=jnp.float32) + b2c_ref[...])
    o3 = o2.reshape(B, 11, 256)
    acc = jnp.zeros((B, 128), jnp.float32)
    for l in range(11):
        acc = acc + jnp.dot(o3[:, l, :], l1w_ref[l],
                            preferred_element_type=jnp.float32)
    xs = jax.nn.relu(acc + l1b_ref[...])
    xs = jnp.dot(xs, l2w_ref[...], preferred_element_type=jnp.float32) + l2b_ref[...]
    inv_sqrt = 1.0 / (1.0 + 1e-5) ** 0.5
    xs = jax.nn.relu(bng_ref[...] * xs * inv_sqrt + bnb_ref[...])
    pv = p_ref[0, 0]
    z = jnp.concatenate([xs * pv, feat_ref[...] * (1.0 - pv)], axis=1)
    z = jax.nn.relu(jnp.dot(z, fw_ref[...],
                            preferred_element_type=jnp.float32) + fb_ref[...])
    out_ref[...] = jnp.dot(z, f2w_ref[...],
                           preferred_element_type=jnp.float32) + f2b_ref[...]


def kernel(x, edge_index, batch, node, di_sim, dr_sim, drug_adj, dis_adj, p,
           W1, b1, W2, b2, conv1_w, conv1_b, conv2_w, conv2_b, lin1_W, lin1_b,
           lin2_W, lin2_b, bn_gamma, bn_beta, lin_di_W, lin_di_b, lin_dr_W,
           lin_dr_b, W3, b3, W4, b4, fcs_W, fcs_b, fcs2_W, fcs2_b):
    f32 = jnp.float32
    i32 = jnp.int32

    x_pad = jnp.concatenate([x, jnp.zeros((NPAD - N, 128), f32)])
    src = jnp.concatenate([edge_index[0].astype(i32),
                           jnp.full((E_PAD - E,), N, i32)]).reshape(NW * NCH, C)
    dst = jnp.concatenate([edge_index[1].astype(i32),
                           jnp.full((E_PAD - E,), NPAD - 8, i32)]).reshape(NW * NCH, C)
    batch_pad = jnp.concatenate([batch.astype(i32),
                                 jnp.full((NPAD - N,), B, i32)]).reshape(1, NPAD)
    ones_deg = jnp.ones((C, DEG_W), f32)
    zeros_deg = jnp.zeros((RPT, DEG_W), f32)
    zeros_agg = jnp.zeros((RPT, F1), f32)
    w1c = conv1_w[:, 0, :].T
    w2r = conv2_w.transpose(2, 1, 0).reshape(640, 256)
    l1w = lin1_W.reshape(256, 11, 128).transpose(1, 0, 2)

    _sc_degree, _sc_aggregate = _sc_kernels()
    deg0, deg1 = _sc_degree(dst, ones_deg, zeros_deg)

    prep_out = pl.pallas_call(
        _tc_prep_body,
        out_shape=(
            jax.ShapeDtypeStruct((NPAD, F1), f32),
            jax.ShapeDtypeStruct((NPAD, F1), f32),
            jax.ShapeDtypeStruct((NPAD, 1), f32),
            jax.ShapeDtypeStruct((NPAD, 1), f32),
            jax.ShapeDtypeStruct((B, F1), f32),
            jax.ShapeDtypeStruct((B, 1), i32),
            jax.ShapeDtypeStruct((B, 1), i32),
        ),
    )(x_pad, W1, deg0, deg1, batch_pad, node.astype(i32),
      di_sim, lin_di_W, lin_di_b.reshape(1, -1), drug_adj,
      dr_sim, lin_dr_W, lin_dr_b.reshape(1, -1), dis_adj,
      W3, b3.reshape(1, -1), W4, b4.reshape(1, -1))
    h1, hs1, dinv, dinv2, feat_out, starts, counts = prep_out

    a10, a11 = _sc_aggregate(src, dst, hs1, zeros_agg)

    x1, h2, hs2 = pl.pallas_call(
        _tc_mid_body,
        out_shape=(
            jax.ShapeDtypeStruct((NPAD, F1), f32),
            jax.ShapeDtypeStruct((NPAD, F1), f32),
            jax.ShapeDtypeStruct((NPAD, F1), f32),
        ),
    )(a10, a11, h1, dinv, dinv2, b1.reshape(1, -1), W2)

    a20, a21 = _sc_aggregate(src, dst, hs2, zeros_agg)

    smem = pltpu.SMEM
    g = pl.pallas_call(
        _tc_pool_body,
        grid=(B,),
        in_specs=[
            pl.BlockSpec(memory_space=smem),
            pl.BlockSpec(memory_space=smem),
            pl.BlockSpec((NPAD, F1), lambda s: (0, 0)),
            pl.BlockSpec((NPAD, F1), lambda s: (0, 0)),
            pl.BlockSpec((NPAD, F1), lambda s: (0, 0)),
            pl.BlockSpec((NPAD, F1), lambda s: (0, 0)),
            pl.BlockSpec((NPAD, 1), lambda s: (0, 0)),
            pl.BlockSpec((NPAD, 1), lambda s: (0, 0)),
            pl.BlockSpec((1, F1), lambda s: (0, 0)),
        ],
        out_specs=pl.BlockSpec((1, K, 64), lambda s: (s, 0, 0)),
        out_shape=jax.ShapeDtypeStruct((B, K, 64), f32),
    )(starts, counts, x1, h2, a20, a21, dinv, dinv2, b2.reshape(1, -1))

    z = pl.pallas_call(
        _tc_head_body,
        out_shape=jax.ShapeDtypeStruct((B, 1), f32),
    )(g, w1c, conv1_b.reshape(1, -1), w2r, conv2_b.reshape(1, -1),
      l1w, lin1_b.reshape(1, -1), lin2_W, lin2_b.reshape(1, -1),
      bn_gamma.reshape(1, -1), bn_beta.reshape(1, -1), feat_out,
      p.reshape(1, 1), fcs_W, fcs_b.reshape(1, -1), fcs2_W,
      fcs2_b.reshape(1, 1))
    return z[:, 0]


# speedup vs baseline: 19.5339x; 1.1471x over previous
import functools

import jax
import jax.numpy as jnp
from jax import lax
from jax.experimental import pallas as pl
from jax.experimental.pallas import tpu as pltpu
from jax.experimental.pallas import tpu_sc as plsc

N = 10000
NPAD = 10496
B = 256
K = 30
W = 256
E = 320000
NC, NS = 2, 16
NW = NC * NS
C = 128
NCH = 80
EPW = NCH * C
E_PAD = NW * EPW
RPT = NPAD // NS
DEG_W = 16
F1 = 32

@functools.cache
def _sc_kernels():
    mesh = plsc.VectorSubcoreMesh(core_axis_name="c", subcore_axis_name="s",
                                  num_cores=NC, num_subcores=NS)

    def _worker_id():
        return lax.axis_index("s") * NC + lax.axis_index("c")

    @functools.partial(
        pl.kernel,
        out_type=(
            jax.ShapeDtypeStruct((NPAD, DEG_W), jnp.float32),
            jax.ShapeDtypeStruct((NPAD, DEG_W), jnp.float32),
        ),
        mesh=mesh,
        scratch_types=[
            pltpu.VMEM((NCH, C), jnp.int32),
            pltpu.VMEM((C, DEG_W), jnp.float32),
            pltpu.VMEM_SHARED((NPAD, DEG_W), jnp.float32),
        ],
        compiler_params=pltpu.CompilerParams(use_tc_tiling_on_sc=False),
    )
    def _sc_degree(dst_hbm, ones_hbm, zeros_hbm, out0, out1,
                   idx_v, ones_v, acc_sh):
        cid = lax.axis_index("c")
        sid = lax.axis_index("s")
        wid = _worker_id()
        pltpu.sync_copy(dst_hbm.at[pl.ds(wid * NCH, NCH), :], idx_v)
        pltpu.sync_copy(ones_hbm, ones_v)
        pltpu.sync_copy(zeros_hbm, acc_sh.at[pl.ds(sid * RPT, RPT), :])
        plsc.subcore_barrier()

        def body(j, carry):
            pltpu.sync_copy(ones_v, acc_sh.at[idx_v.at[j]], add=True)
            return carry

        lax.fori_loop(0, NCH, body, 0)
        plsc.subcore_barrier()

        @pl.when(cid == 0)
        def _():
            pltpu.sync_copy(acc_sh.at[pl.ds(sid * RPT, RPT), :],
                            out0.at[pl.ds(sid * RPT, RPT), :])

        @pl.when(cid == 1)
        def _():
            pltpu.sync_copy(acc_sh.at[pl.ds(sid * RPT, RPT), :],
                            out1.at[pl.ds(sid * RPT, RPT), :])

    @functools.partial(
        pl.kernel,
        out_type=(
            jax.ShapeDtypeStruct((NPAD, F1), jnp.float32),
            jax.ShapeDtypeStruct((NPAD, F1), jnp.float32),
        ),
        mesh=mesh,
        scratch_types=[
            pltpu.VMEM((NCH, C), jnp.int32),
            pltpu.VMEM((NCH, C), jnp.int32),
            pltpu.VMEM((C, F1), jnp.float32),
            pltpu.VMEM((C, F1), jnp.float32),
            pltpu.VMEM_SHARED((NPAD, F1), jnp.float32),
            pltpu.SemaphoreType.DMA,
            pltpu.SemaphoreType.DMA,
        ],
        compiler_params=pltpu.CompilerParams(use_tc_tiling_on_sc=False),
    )
    def _sc_aggregate(src_hbm, dst_hbm, hs_hbm, zeros_hbm, out0, out1,
                      src_v, dst_v, rows0, rows1, acc_sh, sem0, sem1):
        cid = lax.axis_index("c")
        sid = lax.axis_index("s")
        wid = _worker_id()
        pltpu.sync_copy(src_hbm.at[pl.ds(wid * NCH, NCH), :], src_v)
        pltpu.sync_copy(dst_hbm.at[pl.ds(wid * NCH, NCH), :], dst_v)
        pltpu.sync_copy(zeros_hbm, acc_sh.at[pl.ds(sid * RPT, RPT), :])
        plsc.subcore_barrier()

        pltpu.async_copy(hs_hbm.at[src_v.at[0]], rows0, sem0)

        def body(i, carry):
            j0 = 2 * i
            pltpu.async_copy(hs_hbm.at[src_v.at[j0 + 1]], rows1, sem1)
            pltpu.make_async_copy(hs_hbm.at[src_v.at[j0]], rows0, sem0).wait()
            pltpu.sync_copy(rows0, acc_sh.at[dst_v.at[j0]], add=True)

            @pl.when(i < NCH // 2 - 1)
            def _():
                pltpu.async_copy(hs_hbm.at[src_v.at[j0 + 2]], rows0, sem0)

            pltpu.make_async_copy(hs_hbm.at[src_v.at[j0 + 1]], rows1, sem1).wait()
            pltpu.sync_copy(rows1, acc_sh.at[dst_v.at[j0 + 1]], add=True)
            return carry

        lax.fori_loop(0, NCH // 2, body, 0)
        plsc.subcore_barrier()

        @pl.when(cid == 0)
        def _():
            pltpu.sync_copy(acc_sh.at[pl.ds(sid * RPT, RPT), :],
                            out0.at[pl.ds(sid * RPT, RPT), :])

        @pl.when(cid == 1)
        def _():
            pltpu.sync_copy(acc_sh.at[pl.ds(sid * RPT, RPT), :],
                            out1.at[pl.ds(sid * RPT, RPT), :])

    return _sc_degree, _sc_aggregate


def _dense_gcn_pair(emb, adj, W3, b3, W4, b4):
    n = adj.shape[0]
    i0 = lax.broadcasted_iota(jnp.int32, (n, n), 0)
    i1 = lax.broadcasted_iota(jnp.int32, (n, n), 1)
    a = adj + jnp.where(i0 == i1, 1.0, 0.0).astype(jnp.float32)
    deg = jnp.sum(a, axis=1, keepdims=True)
    dinv = lax.rsqrt(deg)
    f = jax.nn.relu(dinv * jnp.dot(a, dinv * jnp.dot(emb, W3),
                                   preferred_element_type=jnp.float32) + b3)
    o = jax.nn.relu(dinv * jnp.dot(a, dinv * jnp.dot(f, W4),
                                   preferred_element_type=jnp.float32) + b4)
    return o


def _tc_prep_body(x_ref, w1_ref, d0_ref, d1_ref, batch_ref, node_ref,
                  disim_ref, ldiW_ref, ldib_ref, dradj_ref,
                  drsim_ref, ldrW_ref, ldrb_ref, disadj_ref,
                  w3_ref, b3_ref, w4_ref, b4_ref,
                  h1_ref, hs1_ref, dinv_ref, dinv2_ref, feat_ref,
                  starts_ref, counts_ref):
    deg = jnp.sum(d0_ref[...] + d1_ref[...], axis=1,
                  keepdims=True) * (1.0 / DEG_W) + 1.0
    dinv = lax.rsqrt(deg)
    dinv_ref[...] = dinv
    dinv2_ref[...] = dinv * dinv
    h1 = jnp.dot(x_ref[...], w1_ref[...], preferred_element_type=jnp.float32)
    h1_ref[...] = h1
    hs1_ref[...] = h1 * dinv

    bcb = jnp.broadcast_to(batch_ref[...], (B, NPAD))
    sseg = lax.broadcasted_iota(jnp.int32, (B, NPAD), 0)
    counts_ref[...] = jnp.sum((bcb == sseg).astype(jnp.int32), axis=1,
                              keepdims=True)
    starts_ref[...] = jnp.sum((bcb < sseg).astype(jnp.int32), axis=1,
                              keepdims=True)

    di_emb = jnp.dot(disim_ref[...], ldiW_ref[...],
                     preferred_element_type=jnp.float32) + ldib_ref[...]
    di_out = _dense_gcn_pair(di_emb, dradj_ref[...], w3_ref[...], b3_ref[...],
                             w4_ref[...], b4_ref[...])
    dr_emb = jnp.dot(drsim_ref[...], ldrW_ref[...],
                     preferred_element_type=jnp.float32) + ldrb_ref[...]
    dr_out = _dense_gcn_pair(dr_emb, disadj_ref[...], w3_ref[...], b3_ref[...],
                             w4_ref[...], b4_ref[...])
    node = node_ref[...]
    oh_di = (lax.broadcasted_iota(jnp.int32, (B, 593), 1)
             == node[:, 0:1]).astype(jnp.float32)
    oh_dr = (lax.broadcasted_iota(jnp.int32, (B, 313), 1)
             == node[:, 1:2]).astype(jnp.float32)
    di_sel = jnp.dot(oh_di, di_out, preferred_element_type=jnp.float32)
    dr_sel = jnp.dot(oh_dr, dr_out, preferred_element_type=jnp.float32)
    feat_ref[...] = jnp.concatenate([di_sel, dr_sel], axis=1)


def _tc_mid_body(a0_ref, a1_ref, h1_ref, dinv_ref, dinv2_ref, b1_ref, w2_ref,
                 x1_ref, h2_ref, hs2_ref):
    dinv = dinv_ref[...]
    x1 = jax.nn.relu(dinv * (a0_ref[...] + a1_ref[...])
                     + dinv2_ref[...] * h1_ref[...] + b1_ref[...])
    x1_ref[...] = x1
    h2 = jnp.dot(x1, w2_ref[...], preferred_element_type=jnp.float32)
    h2_ref[...] = h2
    hs2_ref[...] = h2 * dinv


def _tc_pool_body(starts_ref, counts_ref, x1_ref, h2_ref, a0_ref, a1_ref,
                  dinv_ref, dinv2_ref, b2_ref, g_ref):
    s = pl.program_id(0)
    st = starts_ref[s, 0]
    cnt = counts_ref[s, 0]
    x1_w = x1_ref[pl.ds(st, W), :]
    a_w = a0_ref[pl.ds(st, W), :] + a1_ref[pl.ds(st, W), :]
    h2_w = h2_ref[pl.ds(st, W), :]
    dv = dinv_ref[pl.ds(st, W), :]
    dv2 = dinv2_ref[pl.ds(st, W), :]
    x2_w = jax.nn.relu(dv * a_w + dv2 * h2_w + b2_ref[...])
    keys = x2_w[:, F1 - 1:F1]
    krow = jnp.transpose(keys)
    mi = lax.broadcasted_iota(jnp.int32, (W, W), 0)
    ji = lax.broadcasted_iota(jnp.int32, (W, W), 1)
    better = ((keys > krow) | ((keys == krow) & (mi < ji))) & (mi < cnt)
    rank = jnp.sum(better.astype(jnp.float32), axis=0, keepdims=True)
    pk = lax.broadcasted_iota(jnp.int32, (K, W), 0)
    jk = lax.broadcasted_iota(jnp.int32, (K, W), 1)
    sel = ((pk == rank.astype(jnp.int32)) & (jk < cnt)).astype(jnp.float32)
    h_cat = jnp.concatenate([x1_w, x2_w], axis=1)
    g_ref[0] = jnp.dot(sel, h_cat, preferred_element_type=jnp.float32)


def _tc_head_body(g_ref, w1c_ref, b1c_ref, w2r_ref, b2c_ref, l1w_ref, l1b_ref,
                  l2w_ref, l2b_ref, bng_ref, bnb_ref, feat_ref, p_ref,
                  fw_ref, fb_ref, f2w_ref, f2b_ref, out_ref):
    g2 = g_ref[...].reshape(B * K, 64)
    y = jax.nn.relu(jnp.dot(g2, w1c_ref[...],
                            preferred_element_type=jnp.float32) + b1c_ref[...])
    y4 = y.reshape(B, K // 2, 2, 128)
    pooled = jnp.max(y4, axis=2)
    u = jnp.concatenate([pooled[:, k:k + 11, :] for k in range(5)], axis=2)
    u2 = u.reshape(B * 11, 640)
    o2 = jax.nn.relu(jnp.dot(u2, w2r_ref[...],
                             preferred_element_type=jnp.float32) + b2c_ref[...])
    o3 = o2.reshape(B, 11, 256)
    acc = jnp.zeros((B, 128), jnp.float32)
    for l in range(11):
        acc = acc + jnp.dot(o3[:, l, :], l1w_ref[l],
                            preferred_element_type=jnp.float32)
    xs = jax.nn.relu(acc + l1b_ref[...])
    xs = jnp.dot(xs, l2w_ref[...], preferred_element_type=jnp.float32) + l2b_ref[...]
    inv_sqrt = 1.0 / (1.0 + 1e-5) ** 0.5
    xs = jax.nn.relu(bng_ref[...] * xs * inv_sqrt + bnb_ref[...])
    pv = p_ref[0, 0]
    z = jnp.concatenate([xs * pv, feat_ref[...] * (1.0 - pv)], axis=1)
    z = jax.nn.relu(jnp.dot(z, fw_ref[...],
                            preferred_element_type=jnp.float32) + fb_ref[...])
    out_ref[...] = jnp.dot(z, f2w_ref[...],
                           preferred_element_type=jnp.float32) + f2b_ref[...]


def kernel(x, edge_index, batch, node, di_sim, dr_sim, drug_adj, dis_adj, p,
           W1, b1, W2, b2, conv1_w, conv1_b, conv2_w, conv2_b, lin1_W, lin1_b,
           lin2_W, lin2_b, bn_gamma, bn_beta, lin_di_W, lin_di_b, lin_dr_W,
           lin_dr_b, W3, b3, W4, b4, fcs_W, fcs_b, fcs2_W, fcs2_b):
    f32 = jnp.float32
    i32 = jnp.int32

    x_pad = jnp.concatenate([x, jnp.zeros((NPAD - N, 128), f32)])
    src = jnp.concatenate([edge_index[0].astype(i32),
                           jnp.full((E_PAD - E,), N, i32)]).reshape(NW * NCH, C)
    dst = jnp.concatenate([edge_index[1].astype(i32),
                           jnp.full((E_PAD - E,), NPAD - 8, i32)]).reshape(NW * NCH, C)
    batch_pad = jnp.concatenate([batch.astype(i32),
                                 jnp.full((NPAD - N,), B, i32)]).reshape(1, NPAD)
    ones_deg = jnp.ones((C, DEG_W), f32)
    zeros_deg = jnp.zeros((RPT, DEG_W), f32)
    zeros_agg = jnp.zeros((RPT, F1), f32)
    w1c = conv1_w[:, 0, :].T
    w2r = conv2_w.transpose(2, 1, 0).reshape(640, 256)
    l1w = lin1_W.reshape(256, 11, 128).transpose(1, 0, 2)

    _sc_degree, _sc_aggregate = _sc_kernels()
    deg0, deg1 = _sc_degree(dst, ones_deg, zeros_deg)

    prep_out = pl.pallas_call(
        _tc_prep_body,
        out_shape=(
            jax.ShapeDtypeStruct((NPAD, F1), f32),
            jax.ShapeDtypeStruct((NPAD, F1), f32),
            jax.ShapeDtypeStruct((NPAD, 1), f32),
            jax.ShapeDtypeStruct((NPAD, 1), f32),
            jax.ShapeDtypeStruct((B, F1), f32),
            jax.ShapeDtypeStruct((B, 1), i32),
            jax.ShapeDtypeStruct((B, 1), i32),
        ),
    )(x_pad, W1, deg0, deg1, batch_pad, node.astype(i32),
      di_sim, lin_di_W, lin_di_b.reshape(1, -1), drug_adj,
      dr_sim, lin_dr_W, lin_dr_b.reshape(1, -1), dis_adj,
      W3, b3.reshape(1, -1), W4, b4.reshape(1, -1))
    h1, hs1, dinv, dinv2, feat_out, starts, counts = prep_out

    a10, a11 = _sc_aggregate(src, dst, hs1, zeros_agg)

    x1, h2, hs2 = pl.pallas_call(
        _tc_mid_body,
        out_shape=(
            jax.ShapeDtypeStruct((NPAD, F1), f32),
            jax.ShapeDtypeStruct((NPAD, F1), f32),
            jax.ShapeDtypeStruct((NPAD, F1), f32),
        ),
    )(a10, a11, h1, dinv, dinv2, b1.reshape(1, -1), W2)

    a20, a21 = _sc_aggregate(src, dst, hs2, zeros_agg)

    smem = pltpu.SMEM
    g = pl.pallas_call(
        _tc_pool_body,
        grid=(B,),
        in_specs=[
            pl.BlockSpec(memory_space=smem),
            pl.BlockSpec(memory_space=smem),
            pl.BlockSpec((NPAD, F1), lambda s: (0, 0)),
            pl.BlockSpec((NPAD, F1), lambda s: (0, 0)),
            pl.BlockSpec((NPAD, F1), lambda s: (0, 0)),
            pl.BlockSpec((NPAD, F1), lambda s: (0, 0)),
            pl.BlockSpec((NPAD, 1), lambda s: (0, 0)),
            pl.BlockSpec((NPAD, 1), lambda s: (0, 0)),
            pl.BlockSpec((1, F1), lambda s: (0, 0)),
        ],
        out_specs=pl.BlockSpec((1, K, 64), lambda s: (s, 0, 0)),
        out_shape=jax.ShapeDtypeStruct((B, K, 64), f32),
    )(starts, counts, x1, h2, a20, a21, dinv, dinv2, b2.reshape(1, -1))

    z = pl.pallas_call(
        _tc_head_body,
        out_shape=jax.ShapeDtypeStruct((B, 1), f32),
    )(g, w1c, conv1_b.reshape(1, -1), w2r, conv2_b.reshape(1, -1),
      l1w, lin1_b.reshape(1, -1), lin2_W, lin2_b.reshape(1, -1),
      bn_gamma.reshape(1, -1), bn_beta.reshape(1, -1), feat_out,
      p.reshape(1, 1), fcs_W, fcs_b.reshape(1, -1), fcs2_W,
      fcs2_b.reshape(1, 1))
    return z[:, 0]


# speedup vs baseline: 22.2068x; 1.1368x over previous
import functools

import jax
import jax.numpy as jnp
from jax import lax
from jax.experimental import pallas as pl
from jax.experimental.pallas import tpu as pltpu
from jax.experimental.pallas import tpu_sc as plsc

N = 10000
NPAD = 10496
B = 256
K = 30
W = 256
E = 320000
NC, NS = 2, 16
NW = NC * NS
C = 128
NCH = 80
EPW = NCH * C
E_PAD = NW * EPW
RPT = NPAD // NS
DEG_W = 16
F1 = 32

@functools.cache
def _sc_kernels():
    mesh = plsc.VectorSubcoreMesh(core_axis_name="c", subcore_axis_name="s",
                                  num_cores=NC, num_subcores=NS)

    def _worker_id():
        return lax.axis_index("s") * NC + lax.axis_index("c")

    @functools.partial(
        pl.kernel,
        out_type=(
            jax.ShapeDtypeStruct((NPAD, DEG_W), jnp.float32),
            jax.ShapeDtypeStruct((NPAD, DEG_W), jnp.float32),
        ),
        mesh=mesh,
        scratch_types=[
            pltpu.VMEM((NCH, C), jnp.int32),
            pltpu.VMEM((C, DEG_W), jnp.float32),
            pltpu.VMEM_SHARED((NPAD, DEG_W), jnp.float32),
        ],
        compiler_params=pltpu.CompilerParams(use_tc_tiling_on_sc=False),
    )
    def _sc_degree(dst_hbm, ones_hbm, zeros_hbm, out0, out1,
                   idx_v, ones_v, acc_sh):
        cid = lax.axis_index("c")
        sid = lax.axis_index("s")
        wid = _worker_id()
        pltpu.sync_copy(dst_hbm.at[pl.ds(wid * NCH, NCH), :], idx_v)
        pltpu.sync_copy(ones_hbm, ones_v)
        pltpu.sync_copy(zeros_hbm, acc_sh.at[pl.ds(sid * RPT, RPT), :])
        plsc.subcore_barrier()

        def body(j, carry):
            pltpu.sync_copy(ones_v, acc_sh.at[idx_v.at[j]], add=True)
            return carry

        lax.fori_loop(0, NCH, body, 0)
        plsc.subcore_barrier()

        @pl.when(cid == 0)
        def _():
            pltpu.sync_copy(acc_sh.at[pl.ds(sid * RPT, RPT), :],
                            out0.at[pl.ds(sid * RPT, RPT), :])

        @pl.when(cid == 1)
        def _():
            pltpu.sync_copy(acc_sh.at[pl.ds(sid * RPT, RPT), :],
                            out1.at[pl.ds(sid * RPT, RPT), :])

    @functools.partial(
        pl.kernel,
        out_type=(
            jax.ShapeDtypeStruct((NPAD, F1), jnp.float32),
            jax.ShapeDtypeStruct((NPAD, F1), jnp.float32),
        ),
        mesh=mesh,
        scratch_types=[
            pltpu.VMEM((NCH, C), jnp.int32),
            pltpu.VMEM((NCH, C), jnp.int32),
            pltpu.VMEM((C, F1), jnp.float32),
            pltpu.VMEM((C, F1), jnp.float32),
            pltpu.VMEM_SHARED((NPAD, F1), jnp.float32),
            pltpu.SemaphoreType.DMA,
            pltpu.SemaphoreType.DMA,
        ],
        compiler_params=pltpu.CompilerParams(use_tc_tiling_on_sc=False),
    )
    def _sc_aggregate(src_hbm, dst_hbm, hs_hbm, zeros_hbm, out0, out1,
                      src_v, dst_v, rows0, rows1, acc_sh, sem0, sem1):
        cid = lax.axis_index("c")
        sid = lax.axis_index("s")
        wid = _worker_id()
        pltpu.sync_copy(src_hbm.at[pl.ds(wid * NCH, NCH), :], src_v)
        pltpu.sync_copy(dst_hbm.at[pl.ds(wid * NCH, NCH), :], dst_v)
        pltpu.sync_copy(zeros_hbm, acc_sh.at[pl.ds(sid * RPT, RPT), :])
        plsc.subcore_barrier()

        pltpu.async_copy(hs_hbm.at[src_v.at[0]], rows0, sem0)

        def body(i, carry):
            j0 = 2 * i
            pltpu.async_copy(hs_hbm.at[src_v.at[j0 + 1]], rows1, sem1)
            pltpu.make_async_copy(hs_hbm.at[src_v.at[j0]], rows0, sem0).wait()
            pltpu.sync_copy(rows0, acc_sh.at[dst_v.at[j0]], add=True)

            @pl.when(i < NCH // 2 - 1)
            def _():
                pltpu.async_copy(hs_hbm.at[src_v.at[j0 + 2]], rows0, sem0)

            pltpu.make_async_copy(hs_hbm.at[src_v.at[j0 + 1]], rows1, sem1).wait()
            pltpu.sync_copy(rows1, acc_sh.at[dst_v.at[j0 + 1]], add=True)
            return carry

        lax.fori_loop(0, NCH // 2, body, 0)
        plsc.subcore_barrier()

        @pl.when(cid == 0)
        def _():
            pltpu.sync_copy(acc_sh.at[pl.ds(sid * RPT, RPT), :],
                            out0.at[pl.ds(sid * RPT, RPT), :])

        @pl.when(cid == 1)
        def _():
            pltpu.sync_copy(acc_sh.at[pl.ds(sid * RPT, RPT), :],
                            out1.at[pl.ds(sid * RPT, RPT), :])

    return _sc_degree, _sc_aggregate


def _dense_gcn_pair(emb, adj, W3, b3, W4, b4):
    n = adj.shape[0]
    i0 = lax.broadcasted_iota(jnp.int32, (n, n), 0)
    i1 = lax.broadcasted_iota(jnp.int32, (n, n), 1)
    a = adj + jnp.where(i0 == i1, 1.0, 0.0).astype(jnp.float32)
    deg = jnp.sum(a, axis=1, keepdims=True)
    dinv = lax.rsqrt(deg)
    f = jax.nn.relu(dinv * jnp.dot(a, dinv * jnp.dot(emb, W3),
                                   preferred_element_type=jnp.float32) + b3)
    o = jax.nn.relu(dinv * jnp.dot(a, dinv * jnp.dot(f, W4),
                                   preferred_element_type=jnp.float32) + b4)
    return o


def _tc_prep_body(x_ref, w1_ref, d0_ref, d1_ref, batch_ref, node_ref,
                  disim_ref, ldiW_ref, ldib_ref, dradj_ref,
                  drsim_ref, ldrW_ref, ldrb_ref, disadj_ref,
                  w3_ref, b3_ref, w4_ref, b4_ref,
                  h1_ref, hs1_ref, dinv_ref, dinv2_ref, feat_ref,
                  starts_ref, counts_ref):
    deg = jnp.sum(d0_ref[...] + d1_ref[...], axis=1,
                  keepdims=True) * (1.0 / DEG_W) + 1.0
    dinv = lax.rsqrt(deg)
    dinv_ref[...] = dinv
    dinv2_ref[...] = dinv * dinv
    h1 = jnp.dot(x_ref[...], w1_ref[...], preferred_element_type=jnp.float32)
    h1_ref[...] = h1
    hs1_ref[...] = h1 * dinv

    bcb = jnp.broadcast_to(batch_ref[...], (B, NPAD))
    sseg = lax.broadcasted_iota(jnp.int32, (B, NPAD), 0)
    counts_ref[...] = jnp.sum((bcb == sseg).astype(jnp.int32), axis=1,
                              keepdims=True)
    starts_ref[...] = jnp.sum((bcb < sseg).astype(jnp.int32), axis=1,
                              keepdims=True)

    di_emb = jnp.dot(disim_ref[...], ldiW_ref[...],
                     preferred_element_type=jnp.float32) + ldib_ref[...]
    di_out = _dense_gcn_pair(di_emb, dradj_ref[...], w3_ref[...], b3_ref[...],
                             w4_ref[...], b4_ref[...])
    dr_emb = jnp.dot(drsim_ref[...], ldrW_ref[...],
                     preferred_element_type=jnp.float32) + ldrb_ref[...]
    dr_out = _dense_gcn_pair(dr_emb, disadj_ref[...], w3_ref[...], b3_ref[...],
                             w4_ref[...], b4_ref[...])
    node = node_ref[...]
    oh_di = (lax.broadcasted_iota(jnp.int32, (B, 593), 1)
             == node[:, 0:1]).astype(jnp.float32)
    oh_dr = (lax.broadcasted_iota(jnp.int32, (B, 313), 1)
             == node[:, 1:2]).astype(jnp.float32)
    di_sel = jnp.dot(oh_di, di_out, preferred_element_type=jnp.float32)
    dr_sel = jnp.dot(oh_dr, dr_out, preferred_element_type=jnp.float32)
    feat_ref[...] = jnp.concatenate([di_sel, dr_sel], axis=1)


def _tc_mid_body(a0_ref, a1_ref, h1_ref, dinv_ref, dinv2_ref, b1_ref, w2_ref,
                 x1_ref, h2_ref, hs2_ref):
    dinv = dinv_ref[...]
    x1 = jax.nn.relu(dinv * (a0_ref[...] + a1_ref[...])
                     + dinv2_ref[...] * h1_ref[...] + b1_ref[...])
    x1_ref[...] = x1
    h2 = jnp.dot(x1, w2_ref[...], preferred_element_type=jnp.float32)
    h2_ref[...] = h2
    hs2_ref[...] = h2 * dinv


S_PER = 16


def _tc_pool_body(starts_ref, counts_ref, x1_ref, h2_ref, a0_ref, a1_ref,
                  dinv_ref, dinv2_ref, b2_ref, g_ref):
    sb = pl.program_id(0) * S_PER
    for i in range(S_PER):
        st = starts_ref[sb + i, 0]
        cnt = counts_ref[sb + i, 0]
        x1_w = x1_ref[pl.ds(st, W), :]
        a_w = a0_ref[pl.ds(st, W), :] + a1_ref[pl.ds(st, W), :]
        h2_w = h2_ref[pl.ds(st, W), :]
        dv = dinv_ref[pl.ds(st, W), :]
        dv2 = dinv2_ref[pl.ds(st, W), :]
        x2_w = jax.nn.relu(dv * a_w + dv2 * h2_w + b2_ref[...])
        keys = x2_w[:, F1 - 1:F1]
        krow = jnp.transpose(keys)
        mi = lax.broadcasted_iota(jnp.int32, (W, W), 0)
        ji = lax.broadcasted_iota(jnp.int32, (W, W), 1)
        better = ((keys > krow) | ((keys == krow) & (mi < ji))) & (mi < cnt)
        rank = jnp.sum(better.astype(jnp.float32), axis=0, keepdims=True)
        pk = lax.broadcasted_iota(jnp.int32, (K, W), 0)
        jk = lax.broadcasted_iota(jnp.int32, (K, W), 1)
        sel = ((pk == rank.astype(jnp.int32)) & (jk < cnt)).astype(jnp.float32)
        h_cat = jnp.concatenate([x1_w, x2_w], axis=1)
        g_ref[i] = jnp.dot(sel, h_cat, preferred_element_type=jnp.float32)


def _tc_head_body(g_ref, w1c_ref, b1c_ref, w2r_ref, b2c_ref, l1w_ref, l1b_ref,
                  l2w_ref, l2b_ref, bng_ref, bnb_ref, feat_ref, p_ref,
                  fw_ref, fb_ref, f2w_ref, f2b_ref, out_ref):
    g2 = g_ref[...].reshape(B * K, 64)
    y = jax.nn.relu(jnp.dot(g2, w1c_ref[...],
                            preferred_element_type=jnp.float32) + b1c_ref[...])
    y4 = y.reshape(B, K // 2, 2, 128)
    pooled = jnp.max(y4, axis=2)
    u = jnp.concatenate([pooled[:, k:k + 11, :] for k in range(5)], axis=2)
    u2 = u.reshape(B * 11, 640)
    o2 = jax.nn.relu(jnp.dot(u2, w2r_ref[...],
                             preferred_element_type=jnp.float32) + b2c_ref[...])
    o3 = o2.reshape(B, 11, 256)
    acc = jnp.zeros((B, 128), jnp.float32)
    for l in range(11):
        acc = acc + jnp.dot(o3[:, l, :], l1w_ref[l],
                            preferred_element_type=jnp.float32)
    xs = jax.nn.relu(acc + l1b_ref[...])
    xs = jnp.dot(xs, l2w_ref[...], preferred_element_type=jnp.float32) + l2b_ref[...]
    inv_sqrt = 1.0 / (1.0 + 1e-5) ** 0.5
    xs = jax.nn.relu(bng_ref[...] * xs * inv_sqrt + bnb_ref[...])
    pv = p_ref[0, 0]
    z = jnp.concatenate([xs * pv, feat_ref[...] * (1.0 - pv)], axis=1)
    z = jax.nn.relu(jnp.dot(z, fw_ref[...],
                            preferred_element_type=jnp.float32) + fb_ref[...])
    out_ref[...] = jnp.dot(z, f2w_ref[...],
                           preferred_element_type=jnp.float32) + f2b_ref[...]


def kernel(x, edge_index, batch, node, di_sim, dr_sim, drug_adj, dis_adj, p,
           W1, b1, W2, b2, conv1_w, conv1_b, conv2_w, conv2_b, lin1_W, lin1_b,
           lin2_W, lin2_b, bn_gamma, bn_beta, lin_di_W, lin_di_b, lin_dr_W,
           lin_dr_b, W3, b3, W4, b4, fcs_W, fcs_b, fcs2_W, fcs2_b):
    f32 = jnp.float32
    i32 = jnp.int32

    x_pad = jnp.concatenate([x, jnp.zeros((NPAD - N, 128), f32)])
    src = jnp.concatenate([edge_index[0].astype(i32),
                           jnp.full((E_PAD - E,), N, i32)]).reshape(NW * NCH, C)
    dst = jnp.concatenate([edge_index[1].astype(i32),
                           jnp.full((E_PAD - E,), NPAD - 8, i32)]).reshape(NW * NCH, C)
    batch_pad = jnp.concatenate([batch.astype(i32),
                                 jnp.full((NPAD - N,), B, i32)]).reshape(1, NPAD)
    ones_deg = jnp.ones((C, DEG_W), f32)
    zeros_deg = jnp.zeros((RPT, DEG_W), f32)
    zeros_agg = jnp.zeros((RPT, F1), f32)
    w1c = conv1_w[:, 0, :].T
    w2r = conv2_w.transpose(2, 1, 0).reshape(640, 256)
    l1w = lin1_W.reshape(256, 11, 128).transpose(1, 0, 2)

    _sc_degree, _sc_aggregate = _sc_kernels()
    deg0, deg1 = _sc_degree(dst, ones_deg, zeros_deg)

    prep_out = pl.pallas_call(
        _tc_prep_body,
        out_shape=(
            jax.ShapeDtypeStruct((NPAD, F1), f32),
            jax.ShapeDtypeStruct((NPAD, F1), f32),
            jax.ShapeDtypeStruct((NPAD, 1), f32),
            jax.ShapeDtypeStruct((NPAD, 1), f32),
            jax.ShapeDtypeStruct((B, F1), f32),
            jax.ShapeDtypeStruct((B, 1), i32),
            jax.ShapeDtypeStruct((B, 1), i32),
        ),
    )(x_pad, W1, deg0, deg1, batch_pad, node.astype(i32),
      di_sim, lin_di_W, lin_di_b.reshape(1, -1), drug_adj,
      dr_sim, lin_dr_W, lin_dr_b.reshape(1, -1), dis_adj,
      W3, b3.reshape(1, -1), W4, b4.reshape(1, -1))
    h1, hs1, dinv, dinv2, feat_out, starts, counts = prep_out

    a10, a11 = _sc_aggregate(src, dst, hs1, zeros_agg)

    x1, h2, hs2 = pl.pallas_call(
        _tc_mid_body,
        out_shape=(
            jax.ShapeDtypeStruct((NPAD, F1), f32),
            jax.ShapeDtypeStruct((NPAD, F1), f32),
            jax.ShapeDtypeStruct((NPAD, F1), f32),
        ),
    )(a10, a11, h1, dinv, dinv2, b1.reshape(1, -1), W2)

    a20, a21 = _sc_aggregate(src, dst, hs2, zeros_agg)

    smem = pltpu.SMEM
    g = pl.pallas_call(
        _tc_pool_body,
        grid=(B // S_PER,),
        in_specs=[
            pl.BlockSpec(memory_space=smem),
            pl.BlockSpec(memory_space=smem),
            pl.BlockSpec((NPAD, F1), lambda s: (0, 0)),
            pl.BlockSpec((NPAD, F1), lambda s: (0, 0)),
            pl.BlockSpec((NPAD, F1), lambda s: (0, 0)),
            pl.BlockSpec((NPAD, F1), lambda s: (0, 0)),
            pl.BlockSpec((NPAD, 1), lambda s: (0, 0)),
            pl.BlockSpec((NPAD, 1), lambda s: (0, 0)),
            pl.BlockSpec((1, F1), lambda s: (0, 0)),
        ],
        out_specs=pl.BlockSpec((S_PER, K, 64), lambda s: (s, 0, 0)),
        out_shape=jax.ShapeDtypeStruct((B, K, 64), f32),
    )(starts, counts, x1, h2, a20, a21, dinv, dinv2, b2.reshape(1, -1))

    z = pl.pallas_call(
        _tc_head_body,
        out_shape=jax.ShapeDtypeStruct((B, 1), f32),
    )(g, w1c, conv1_b.reshape(1, -1), w2r, conv2_b.reshape(1, -1),
      l1w, lin1_b.reshape(1, -1), lin2_W, lin2_b.reshape(1, -1),
      bn_gamma.reshape(1, -1), bn_beta.reshape(1, -1), feat_out,
      p.reshape(1, 1), fcs_W, fcs_b.reshape(1, -1), fcs2_W,
      fcs2_b.reshape(1, 1))
    return z[:, 0]


# speedup vs baseline: 22.6823x; 1.0214x over previous
import functools

import jax
import jax.numpy as jnp
from jax import lax
from jax.experimental import pallas as pl
from jax.experimental.pallas import tpu as pltpu
from jax.experimental.pallas import tpu_sc as plsc

N = 10000
NPAD = 10496
B = 256
K = 30
W = 256
E = 320000
NC, NS = 2, 16
NW = NC * NS
C = 128
NCH = 80
EPW = NCH * C
E_PAD = NW * EPW
RPT = NPAD // NS
DEG_W = 16
F1 = 32

@functools.cache
def _sc_kernels():
    mesh = plsc.VectorSubcoreMesh(core_axis_name="c", subcore_axis_name="s",
                                  num_cores=NC, num_subcores=NS)

    def _worker_id():
        return lax.axis_index("s") * NC + lax.axis_index("c")

    @functools.partial(
        pl.kernel,
        out_type=(
            jax.ShapeDtypeStruct((NPAD, DEG_W), jnp.float32),
            jax.ShapeDtypeStruct((NPAD, DEG_W), jnp.float32),
        ),
        mesh=mesh,
        scratch_types=[
            pltpu.VMEM((NCH, C), jnp.int32),
            pltpu.VMEM((C, DEG_W), jnp.float32),
            pltpu.VMEM_SHARED((NPAD, DEG_W), jnp.float32),
        ],
        compiler_params=pltpu.CompilerParams(use_tc_tiling_on_sc=False),
    )
    def _sc_degree(dst_hbm, ones_hbm, zeros_hbm, out0, out1,
                   idx_v, ones_v, acc_sh):
        cid = lax.axis_index("c")
        sid = lax.axis_index("s")
        wid = _worker_id()
        pltpu.sync_copy(dst_hbm.at[pl.ds(wid * NCH, NCH), :], idx_v)
        pltpu.sync_copy(ones_hbm, ones_v)
        pltpu.sync_copy(zeros_hbm, acc_sh.at[pl.ds(sid * RPT, RPT), :])
        plsc.subcore_barrier()

        def body(j, carry):
            pltpu.sync_copy(ones_v, acc_sh.at[idx_v.at[j]], add=True)
            return carry

        lax.fori_loop(0, NCH, body, 0)
        plsc.subcore_barrier()

        @pl.when(cid == 0)
        def _():
            pltpu.sync_copy(acc_sh.at[pl.ds(sid * RPT, RPT), :],
                            out0.at[pl.ds(sid * RPT, RPT), :])

        @pl.when(cid == 1)
        def _():
            pltpu.sync_copy(acc_sh.at[pl.ds(sid * RPT, RPT), :],
                            out1.at[pl.ds(sid * RPT, RPT), :])

    @functools.partial(
        pl.kernel,
        out_type=(
            jax.ShapeDtypeStruct((NPAD, F1), jnp.float32),
            jax.ShapeDtypeStruct((NPAD, F1), jnp.float32),
        ),
        mesh=mesh,
        scratch_types=[
            pltpu.VMEM((NCH, C), jnp.int32),
            pltpu.VMEM((NCH, C), jnp.int32),
            [pltpu.VMEM((C, F1), jnp.float32) for _ in range(8)],
            pltpu.VMEM_SHARED((NPAD, F1), jnp.float32),
            [pltpu.SemaphoreType.DMA for _ in range(8)],
            [pltpu.SemaphoreType.DMA for _ in range(8)],
        ],
        compiler_params=pltpu.CompilerParams(use_tc_tiling_on_sc=False),
    )
    def _sc_aggregate(src_hbm, dst_hbm, hs_hbm, zeros_hbm, out0, out1,
                      src_v, dst_v, rows, acc_sh, gsem, ssem):
        cid = lax.axis_index("c")
        sid = lax.axis_index("s")
        wid = _worker_id()
        pltpu.sync_copy(src_hbm.at[pl.ds(wid * NCH, NCH), :], src_v)
        pltpu.sync_copy(dst_hbm.at[pl.ds(wid * NCH, NCH), :], dst_v)
        pltpu.sync_copy(zeros_hbm, acc_sh.at[pl.ds(sid * RPT, RPT), :])
        plsc.subcore_barrier()

        for b in range(8):
            pltpu.async_copy(hs_hbm.at[src_v.at[b]], rows[b], gsem[b])

        def body(i, carry):
            for b in range(8):
                c = 8 * i + b
                bp = (b + 4) % 8
                pltpu.make_async_copy(hs_hbm.at[src_v.at[c]],
                                      rows[b], gsem[b]).wait()
                pltpu.async_copy(rows[b], acc_sh.at[dst_v.at[c]], ssem[b],
                                 add=True)

                def prefetch():
                    pltpu.make_async_copy(rows[bp], acc_sh.at[dst_v.at[c - 4]],
                                          ssem[bp]).wait()
                    pltpu.async_copy(hs_hbm.at[src_v.at[c + 4]],
                                     rows[bp], gsem[bp])

                if b < 4:
                    pl.when(i > 0)(prefetch)
                else:
                    pl.when(i < NCH // 8 - 1)(prefetch)
            return carry

        lax.fori_loop(0, NCH // 8, body, 0)
        for b in range(8):
            pltpu.make_async_copy(rows[b], acc_sh.at[dst_v.at[NCH - 8 + b]],
                                  ssem[b]).wait()
        plsc.subcore_barrier()

        @pl.when(cid == 0)
        def _():
            pltpu.sync_copy(acc_sh.at[pl.ds(sid * RPT, RPT), :],
                            out0.at[pl.ds(sid * RPT, RPT), :])

        @pl.when(cid == 1)
        def _():
            pltpu.sync_copy(acc_sh.at[pl.ds(sid * RPT, RPT), :],
                            out1.at[pl.ds(sid * RPT, RPT), :])

    return _sc_degree, _sc_aggregate


def _dense_gcn_pair(emb, adj, W3, b3, W4, b4):
    n = adj.shape[0]
    i0 = lax.broadcasted_iota(jnp.int32, (n, n), 0)
    i1 = lax.broadcasted_iota(jnp.int32, (n, n), 1)
    a = adj + jnp.where(i0 == i1, 1.0, 0.0).astype(jnp.float32)
    deg = jnp.sum(a, axis=1, keepdims=True)
    dinv = lax.rsqrt(deg)
    f = jax.nn.relu(dinv * jnp.dot(a, dinv * jnp.dot(emb, W3),
                                   preferred_element_type=jnp.float32) + b3)
    o = jax.nn.relu(dinv * jnp.dot(a, dinv * jnp.dot(f, W4),
                                   preferred_element_type=jnp.float32) + b4)
    return o


def _tc_prep_body(x_ref, w1_ref, d0_ref, d1_ref, batch_ref, node_ref,
                  disim_ref, ldiW_ref, ldib_ref, dradj_ref,
                  drsim_ref, ldrW_ref, ldrb_ref, disadj_ref,
                  w3_ref, b3_ref, w4_ref, b4_ref,
                  h1_ref, hs1_ref, dinv_ref, dinv2_ref, feat_ref,
                  starts_ref, counts_ref):
    deg = jnp.sum(d0_ref[...] + d1_ref[...], axis=1,
                  keepdims=True) * (1.0 / DEG_W) + 1.0
    dinv = lax.rsqrt(deg)
    dinv_ref[...] = dinv
    dinv2_ref[...] = dinv * dinv
    h1 = jnp.dot(x_ref[...], w1_ref[...], preferred_element_type=jnp.float32)
    h1_ref[...] = h1
    hs1_ref[...] = h1 * dinv

    bcb = jnp.broadcast_to(batch_ref[...], (B, NPAD))
    sseg = lax.broadcasted_iota(jnp.int32, (B, NPAD), 0)
    counts_ref[...] = jnp.sum((bcb == sseg).astype(jnp.int32), axis=1,
                              keepdims=True)
    starts_ref[...] = jnp.sum((bcb < sseg).astype(jnp.int32), axis=1,
                              keepdims=True)

    di_emb = jnp.dot(disim_ref[...], ldiW_ref[...],
                     preferred_element_type=jnp.float32) + ldib_ref[...]
    di_out = _dense_gcn_pair(di_emb, dradj_ref[...], w3_ref[...], b3_ref[...],
                             w4_ref[...], b4_ref[...])
    dr_emb = jnp.dot(drsim_ref[...], ldrW_ref[...],
                     preferred_element_type=jnp.float32) + ldrb_ref[...]
    dr_out = _dense_gcn_pair(dr_emb, disadj_ref[...], w3_ref[...], b3_ref[...],
                             w4_ref[...], b4_ref[...])
    node = node_ref[...]
    oh_di = (lax.broadcasted_iota(jnp.int32, (B, 593), 1)
             == node[:, 0:1]).astype(jnp.float32)
    oh_dr = (lax.broadcasted_iota(jnp.int32, (B, 313), 1)
             == node[:, 1:2]).astype(jnp.float32)
    di_sel = jnp.dot(oh_di, di_out, preferred_element_type=jnp.float32)
    dr_sel = jnp.dot(oh_dr, dr_out, preferred_element_type=jnp.float32)
    feat_ref[...] = jnp.concatenate([di_sel, dr_sel], axis=1)


def _tc_mid_body(a0_ref, a1_ref, h1_ref, dinv_ref, dinv2_ref, b1_ref, w2_ref,
                 x1_ref, h2_ref, hs2_ref):
    dinv = dinv_ref[...]
    x1 = jax.nn.relu(dinv * (a0_ref[...] + a1_ref[...])
                     + dinv2_ref[...] * h1_ref[...] + b1_ref[...])
    x1_ref[...] = x1
    h2 = jnp.dot(x1, w2_ref[...], preferred_element_type=jnp.float32)
    h2_ref[...] = h2
    hs2_ref[...] = h2 * dinv


S_PER = 16


def _tc_pool_body(starts_ref, counts_ref, x1_ref, h2_ref, a0_ref, a1_ref,
                  dinv_ref, dinv2_ref, b2_ref, g_ref):
    sb = pl.program_id(0) * S_PER
    for i in range(S_PER):
        st = starts_ref[sb + i, 0]
        cnt = counts_ref[sb + i, 0]
        x1_w = x1_ref[pl.ds(st, W), :]
        a_w = a0_ref[pl.ds(st, W), :] + a1_ref[pl.ds(st, W), :]
        h2_w = h2_ref[pl.ds(st, W), :]
        dv = dinv_ref[pl.ds(st, W), :]
        dv2 = dinv2_ref[pl.ds(st, W), :]
        x2_w = jax.nn.relu(dv * a_w + dv2 * h2_w + b2_ref[...])
        keys = x2_w[:, F1 - 1:F1]
        krow = jnp.transpose(keys)
        mi = lax.broadcasted_iota(jnp.int32, (W, W), 0)
        ji = lax.broadcasted_iota(jnp.int32, (W, W), 1)
        better = ((keys > krow) | ((keys == krow) & (mi < ji))) & (mi < cnt)
        rank = jnp.sum(better.astype(jnp.float32), axis=0, keepdims=True)
        pk = lax.broadcasted_iota(jnp.int32, (K, W), 0)
        jk = lax.broadcasted_iota(jnp.int32, (K, W), 1)
        sel = ((pk == rank.astype(jnp.int32)) & (jk < cnt)).astype(jnp.float32)
        h_cat = jnp.concatenate([x1_w, x2_w], axis=1)
        g_ref[i] = jnp.dot(sel, h_cat, preferred_element_type=jnp.float32)


def _tc_head_body(g_ref, w1c_ref, b1c_ref, w2r_ref, b2c_ref, l1w_ref, l1b_ref,
                  l2w_ref, l2b_ref, bng_ref, bnb_ref, feat_ref, p_ref,
                  fw_ref, fb_ref, f2w_ref, f2b_ref, out_ref):
    g2 = g_ref[...].reshape(B * K, 64)
    y = jax.nn.relu(jnp.dot(g2, w1c_ref[...],
                            preferred_element_type=jnp.float32) + b1c_ref[...])
    y4 = y.reshape(B, K // 2, 2, 128)
    pooled = jnp.max(y4, axis=2)
    u = jnp.concatenate([pooled[:, k:k + 11, :] for k in range(5)], axis=2)
    u2 = u.reshape(B * 11, 640)
    o2 = jax.nn.relu(jnp.dot(u2, w2r_ref[...],
                             preferred_element_type=jnp.float32) + b2c_ref[...])
    o3 = o2.reshape(B, 11, 256)
    acc = jnp.zeros((B, 128), jnp.float32)
    for l in range(11):
        acc = acc + jnp.dot(o3[:, l, :], l1w_ref[l],
                            preferred_element_type=jnp.float32)
    xs = jax.nn.relu(acc + l1b_ref[...])
    xs = jnp.dot(xs, l2w_ref[...], preferred_element_type=jnp.float32) + l2b_ref[...]
    inv_sqrt = 1.0 / (1.0 + 1e-5) ** 0.5
    xs = jax.nn.relu(bng_ref[...] * xs * inv_sqrt + bnb_ref[...])
    pv = p_ref[0, 0]
    z = jnp.concatenate([xs * pv, feat_ref[...] * (1.0 - pv)], axis=1)
    z = jax.nn.relu(jnp.dot(z, fw_ref[...],
                            preferred_element_type=jnp.float32) + fb_ref[...])
    out_ref[...] = jnp.dot(z, f2w_ref[...],
                           preferred_element_type=jnp.float32) + f2b_ref[...]


def kernel(x, edge_index, batch, node, di_sim, dr_sim, drug_adj, dis_adj, p,
           W1, b1, W2, b2, conv1_w, conv1_b, conv2_w, conv2_b, lin1_W, lin1_b,
           lin2_W, lin2_b, bn_gamma, bn_beta, lin_di_W, lin_di_b, lin_dr_W,
           lin_dr_b, W3, b3, W4, b4, fcs_W, fcs_b, fcs2_W, fcs2_b):
    f32 = jnp.float32
    i32 = jnp.int32

    x_pad = jnp.concatenate([x, jnp.zeros((NPAD - N, 128), f32)])
    src = jnp.concatenate([edge_index[0].astype(i32),
                           jnp.full((E_PAD - E,), N, i32)]).reshape(NW * NCH, C)
    dst = jnp.concatenate([edge_index[1].astype(i32),
                           jnp.full((E_PAD - E,), NPAD - 8, i32)]).reshape(NW * NCH, C)
    batch_pad = jnp.concatenate([batch.astype(i32),
                                 jnp.full((NPAD - N,), B, i32)]).reshape(1, NPAD)
    ones_deg = jnp.ones((C, DEG_W), f32)
    zeros_deg = jnp.zeros((RPT, DEG_W), f32)
    zeros_agg = jnp.zeros((RPT, F1), f32)
    w1c = conv1_w[:, 0, :].T
    w2r = conv2_w.transpose(2, 1, 0).reshape(640, 256)
    l1w = lin1_W.reshape(256, 11, 128).transpose(1, 0, 2)

    _sc_degree, _sc_aggregate = _sc_kernels()
    deg0, deg1 = _sc_degree(dst, ones_deg, zeros_deg)

    prep_out = pl.pallas_call(
        _tc_prep_body,
        out_shape=(
            jax.ShapeDtypeStruct((NPAD, F1), f32),
            jax.ShapeDtypeStruct((NPAD, F1), f32),
            jax.ShapeDtypeStruct((NPAD, 1), f32),
            jax.ShapeDtypeStruct((NPAD, 1), f32),
            jax.ShapeDtypeStruct((B, F1), f32),
            jax.ShapeDtypeStruct((B, 1), i32),
            jax.ShapeDtypeStruct((B, 1), i32),
        ),
    )(x_pad, W1, deg0, deg1, batch_pad, node.astype(i32),
      di_sim, lin_di_W, lin_di_b.reshape(1, -1), drug_adj,
      dr_sim, lin_dr_W, lin_dr_b.reshape(1, -1), dis_adj,
      W3, b3.reshape(1, -1), W4, b4.reshape(1, -1))
    h1, hs1, dinv, dinv2, feat_out, starts, counts = prep_out

    a10, a11 = _sc_aggregate(src, dst, hs1, zeros_agg)

    x1, h2, hs2 = pl.pallas_call(
        _tc_mid_body,
        out_shape=(
            jax.ShapeDtypeStruct((NPAD, F1), f32),
            jax.ShapeDtypeStruct((NPAD, F1), f32),
            jax.ShapeDtypeStruct((NPAD, F1), f32),
        ),
    )(a10, a11, h1, dinv, dinv2, b1.reshape(1, -1), W2)

    a20, a21 = _sc_aggregate(src, dst, hs2, zeros_agg)

    smem = pltpu.SMEM
    g = pl.pallas_call(
        _tc_pool_body,
        grid=(B // S_PER,),
        in_specs=[
            pl.BlockSpec(memory_space=smem),
            pl.BlockSpec(memory_space=smem),
            pl.BlockSpec((NPAD, F1), lambda s: (0, 0)),
            pl.BlockSpec((NPAD, F1), lambda s: (0, 0)),
            pl.BlockSpec((NPAD, F1), lambda s: (0, 0)),
            pl.BlockSpec((NPAD, F1), lambda s: (0, 0)),
            pl.BlockSpec((NPAD, 1), lambda s: (0, 0)),
            pl.BlockSpec((NPAD, 1), lambda s: (0, 0)),
            pl.BlockSpec((1, F1), lambda s: (0, 0)),
        ],
        out_specs=pl.BlockSpec((S_PER, K, 64), lambda s: (s, 0, 0)),
        out_shape=jax.ShapeDtypeStruct((B, K, 64), f32),
    )(starts, counts, x1, h2, a20, a21, dinv, dinv2, b2.reshape(1, -1))

    z = pl.pallas_call(
        _tc_head_body,
        out_shape=jax.ShapeDtypeStruct((B, 1), f32),
    )(g, w1c, conv1_b.reshape(1, -1), w2r, conv2_b.reshape(1, -1),
      l1w, lin1_b.reshape(1, -1), lin2_W, lin2_b.reshape(1, -1),
      bn_gamma.reshape(1, -1), bn_beta.reshape(1, -1), feat_out,
      p.reshape(1, 1), fcs_W, fcs_b.reshape(1, -1), fcs2_W,
      fcs2_b.reshape(1, 1))
    return z[:, 0]
